# baseline (speedup 1.0000x reference)
import jax
import jax.numpy as jnp
from jax import lax
from jax.experimental import pallas as pl
from jax.experimental.pallas import tpu as pltpu
from jax.experimental.pallas import tpu_sc as plsc

DIM = 64
FIELDS = 26
SLICE = 16
NSL = DIM // SLICE

NC = 2
NS = 16
NW = NC * NS

BT = 128
BT_PER_W = 4
NBLK = FIELDS * BT_PER_W


def _body(idx_hbm, table_hbm, out_hbm, idx_v, G0, G1, S0, S1,
          gsem0, gsem1, wsem0, wsem1):
    wid = lax.axis_index("s") * NC + lax.axis_index("c")

    pltpu.sync_copy(idx_hbm.at[wid], idx_v)

    G = (G0, G1)
    S = (S0, S1)
    gsems = (gsem0, gsem1)
    wsems = (wsem0, wsem1)

    def fire_gather(k, buf, sem):
        for g in range(NSL):
            pltpu.async_copy(table_hbm.at[idx_v.at[k, g]], buf.at[g], sem)

    def wait_gather(buf, sem):
        for g in range(NSL):
            pltpu.make_async_copy(
                table_hbm.at[pl.ds(0, BT)], buf.at[g], sem).wait()

    def fire_write(k, buf, sem):
        f = k // BT_PER_W
        bt = wid * BT_PER_W + (k % BT_PER_W)
        for db in range(8):
            pltpu.async_copy(buf.at[db], out_hbm.at[f, db, bt], sem)

    def wait_write(buf, sem):
        pltpu.make_async_copy(out_hbm.at[0, :, 0], buf, sem).wait()

    iota = lax.iota(jnp.int32, 16)
    zv = iota * 0

    steps = [(g, dd, j)
             for g in range(NSL)
             for dd in range(SLICE)
             for j in range(BT // 16)]
    LEAD = 8

    def transpose(gbuf, sbuf):
        def load(t):
            g, dd, j = steps[t]
            return plsc.load_gather(gbuf, [zv + g, iota + j * 16, zv + dd])

        def store(t, vals):
            g, dd, j = steps[t]
            d = g * SLICE + dd
            sbuf[d // 8, d % 8, pl.ds(j * 16, 16)] = vals

        pending = [load(t) for t in range(LEAD)]
        for t in range(len(steps)):
            if t + LEAD < len(steps):
                pending.append(load(t + LEAD))
            store(t, pending[t])

    fire_gather(0, G0, gsem0)
    fire_gather(1, G1, gsem1)

    def body(i, carry):
        for b in range(2):
            k = i * 2 + b

            @pl.when(k >= 2)
            def _():
                wait_write(S[b], wsems[b])

            wait_gather(G[b], gsems[b])
            transpose(G[b], S[b])
            fire_write(k, S[b], wsems[b])

            @pl.when(k <= NBLK - 3)
            def _():
                fire_gather(k + 2, G[b], gsems[b])

        return carry

    lax.fori_loop(0, NBLK // 2, body, 0)

    for b in range(2):
        wait_write(S[b], wsems[b])


def kernel(x, table):
    batch, fields = x.shape
    table16 = table.reshape(-1, SLICE)

    a = (jnp.transpose(x, (1, 0))
         .reshape(fields, NW, BT_PER_W, BT)
         .transpose(1, 0, 2, 3)
         .reshape(NW, NBLK, BT))
    idx4 = (a[:, :, None, :] * NSL
            + jnp.arange(NSL, dtype=jnp.int32)[None, None, :, None])

    out5 = pl.kernel(
        _body,
        out_type=jax.ShapeDtypeStruct(
            (fields, DIM // 8, batch // BT, 8, BT), jnp.float32),
        mesh=plsc.VectorSubcoreMesh(core_axis_name="c", subcore_axis_name="s"),
        compiler_params=pltpu.CompilerParams(
            use_tc_tiling_on_sc=False, needs_layout_passes=False),
        scratch_types=[
            pltpu.VMEM((NBLK, NSL, BT), jnp.int32),
            pltpu.VMEM((NSL, BT, SLICE), jnp.float32),
            pltpu.VMEM((NSL, BT, SLICE), jnp.float32),
            pltpu.VMEM((8, 8, BT), jnp.float32),
            pltpu.VMEM((8, 8, BT), jnp.float32),
            pltpu.SemaphoreType.DMA,
            pltpu.SemaphoreType.DMA,
            pltpu.SemaphoreType.DMA,
            pltpu.SemaphoreType.DMA,
        ],
    )(idx4, table16)
    return jnp.transpose(out5, (2, 4, 0, 1, 3)).reshape(batch, fields, DIM)


# speedup vs baseline: 1.0424x; 1.0424x over previous
import jax
import jax.numpy as jnp
from jax import lax
from jax.experimental import pallas as pl
from jax.experimental.pallas import tpu as pltpu
from jax.experimental.pallas import tpu_sc as plsc

DIM = 64

NC = 2
NS = 16
NW = NC * NS

IDXW = 128
GPC = 2
CHUNK = IDXW * GPC
WIDE = 128


def _gather_body(idx_hbm, table_hbm, out_hbm, idx_v, rows0, rows1, sem0, sem1):
    wid = lax.axis_index("s") * NC + lax.axis_index("c")
    rows_per_w = idx_hbm.shape[1] * idx_hbm.shape[2]
    n_chunks = rows_per_w // CHUNK
    base = wid * rows_per_w

    pltpu.sync_copy(idx_hbm.at[wid], idx_v)

    bufs = (rows0, rows1)
    sems = (sem0, sem1)

    def fire(c, buf, sem):
        for j in range(GPC):
            pltpu.async_copy(
                table_hbm.at[idx_v.at[c * GPC + j]],
                buf.at[pl.ds(j * IDXW, IDXW)],
                sem,
            )

    def drain(buf, sem):
        pltpu.make_async_copy(table_hbm.at[pl.ds(0, CHUNK)], buf, sem).wait()

    fire(0, rows0, sem0)
    fire(1, rows1, sem1)

    def body(i, carry):
        for b in range(2):
            c = i * 2 + b
            drain(bufs[b], sems[b])
            pltpu.sync_copy(bufs[b].at[:, pl.ds(0, DIM)],
                            out_hbm.at[pl.ds(base + c * CHUNK, CHUNK)])
            fire(c + 2, bufs[b], sems[b])
        return carry

    lax.fori_loop(0, (n_chunks - 2) // 2, body, 0)

    for b in range(2):
        c = n_chunks - 2 + b
        drain(bufs[b], sems[b])
        pltpu.sync_copy(bufs[b].at[:, pl.ds(0, DIM)],
                        out_hbm.at[pl.ds(base + c * CHUNK, CHUNK)])


def kernel(x, table):
    batch, fields = x.shape
    total = batch * fields
    rows_per_w = total // NW
    idx3 = x.reshape(NW, rows_per_w // IDXW, IDXW)
    table_wide = jnp.pad(table, ((0, 0), (0, 128 - DIM)))

    out = pl.kernel(
        _gather_body,
        out_type=jax.ShapeDtypeStruct((total, DIM), jnp.float32),
        mesh=plsc.VectorSubcoreMesh(core_axis_name="c", subcore_axis_name="s"),
        compiler_params=pltpu.CompilerParams(use_tc_tiling_on_sc=False),
        scratch_types=[
            pltpu.VMEM((rows_per_w // IDXW, IDXW), jnp.int32),
            pltpu.VMEM((CHUNK, WIDE), jnp.float32),
            pltpu.VMEM((CHUNK, WIDE), jnp.float32),
            pltpu.SemaphoreType.DMA,
            pltpu.SemaphoreType.DMA,
        ],
    )(idx3, table_wide)
    return out.reshape(batch, fields, DIM)
